# baseline (speedup 1.0000x reference)
import functools

import jax
import jax.numpy as jnp
from jax import lax
from jax.experimental import pallas as pl
from jax.experimental.pallas import tpu as pltpu
from jax.experimental.pallas import tpu_sc as plsc

_B = 4096
_F = 26
_D = 16
_V = 2_600_000
_LANES = 16


_T_BLK = 65536
_SB = _T_BLK // 16
_TSH = _T_BLK.bit_length() - 1
_SSH = _SB.bit_length() - 1


def _transpose_body(et_ref, out_ref):
    et = et_ref[...]
    lo = jax.lax.bitcast_convert_type(
        et[:8].astype(jnp.bfloat16), jnp.uint16).astype(jnp.uint32)
    hi = jax.lax.bitcast_convert_type(
        et[8:].astype(jnp.bfloat16), jnp.uint16).astype(jnp.uint32)
    p = (lo | (hi << 16)).astype(jnp.int32)
    u = jnp.concatenate(
        [p[:, s * _SB:(s + 1) * _SB] for s in range(16)], axis=0)
    out_ref[...] = u.T


def _retile(et):
    grid = (_V + _T_BLK - 1) // _T_BLK
    return pl.pallas_call(
        _transpose_body,
        grid=(grid,),
        in_specs=[pl.BlockSpec((_D, _T_BLK), lambda i: (0, i))],
        out_specs=pl.BlockSpec((_SB, 128), lambda i: (i, 0)),
        out_shape=jax.ShapeDtypeStruct((grid * _SB, 128), jnp.int32),
    )(et)


def _build_gather():
    mesh = plsc.VectorSubcoreMesh(core_axis_name="c", subcore_axis_name="s")

    @functools.partial(
        pl.kernel,
        mesh=mesh,
        out_type=jax.ShapeDtypeStruct((_F, _D, _B), jnp.float32),
        compiler_params=pltpu.CompilerParams(needs_layout_passes=False),
        scratch_types=[
            pltpu.VMEM((_F, 128), jnp.int32),
            pltpu.VMEM((_F, 128), jnp.int32),
            pltpu.VMEM((6, 128), jnp.int32),
            pltpu.VMEM((6, 128, 128), jnp.int32),
            pltpu.VMEM((2, _D, 128), jnp.float32),
            pltpu.SemaphoreType.DMA((6,)),
            pltpu.SemaphoreType.DMA((2,)),
        ],
    )
    def k(xt_hbm, ct_hbm, table_hbm, out_hbm,
          xv, cv, bidx_v, blocks_v, outt_v, gsem, osem):
        wid = lax.axis_index("s") * 2 + lax.axis_index("c")
        b0 = wid * 128

        pltpu.sync_copy(xt_hbm.at[:, pl.ds(b0, 128)], xv)
        pltpu.sync_copy(ct_hbm.at[:, pl.ds(b0, 128)], cv)

        def compute_bidx(f, sel):
            def bb(j, carry):
                st = pl.multiple_of(j * _LANES, _LANES)
                xx = xv[f, pl.ds(st, _LANES)]
                bidx_v[sel, pl.ds(st, _LANES)] = (
                    ((xx >> _TSH) << _SSH) | (xx & (_SB - 1)))
                return carry
            lax.fori_loop(0, 128 // _LANES, bb, 0)

        def fire(f, sel):
            compute_bidx(f, sel)
            pltpu.async_copy(
                table_hbm.at[bidx_v.at[sel]], blocks_v.at[sel], gsem.at[sel])

        def gwait(sel):
            pltpu.make_async_copy(
                table_hbm.at[bidx_v.at[sel]], blocks_v.at[sel], gsem.at[sel]
            ).wait()

        def owait(f, sel):
            pltpu.make_async_copy(
                outt_v.at[sel],
                out_hbm.at[f, :, pl.ds(b0, 128)],
                osem.at[sel],
            ).wait()

        for p in range(5):
            fire(p, p)
        lanes = lax.iota(jnp.int32, _LANES)

        def body(f, carry):
            sel = lax.rem(f, 6)
            osel = lax.rem(f, 2)

            @pl.when(f < _F - 5)
            def _():
                fire(f + 5, lax.rem(f + 5, 6))

            gwait(sel)

            @pl.when(f >= 2)
            def _():
                owait(f - 2, osel)

            def kb(kk, carry2):
                st = pl.multiple_of(kk * _LANES, _LANES)
                x16 = xv[f, pl.ds(st, _LANES)]
                c16 = cv[f, pl.ds(st, _LANES)]
                off16 = ((x16 >> _SSH) & 15) << 3
                row16 = lanes + st
                sel16 = jnp.full((_LANES,), sel, jnp.int32)
                for dd in range(8):
                    v32 = plsc.load_gather(
                        blocks_v, [sel16, row16, off16 + dd])
                    flo = plsc.bitcast(v32 << 16, jnp.float32)
                    fhi = plsc.bitcast(v32 & jnp.int32(-65536), jnp.float32)
                    outt_v[osel, dd, pl.ds(st, _LANES)] = jnp.where(
                        c16 >= dd, flo, 0.0)
                    outt_v[osel, dd + 8, pl.ds(st, _LANES)] = jnp.where(
                        c16 >= dd + 8, fhi, 0.0)
                return carry2

            lax.fori_loop(0, 128 // _LANES, kb, 0)

            pltpu.async_copy(
                outt_v.at[osel], out_hbm.at[f, :, pl.ds(b0, 128)],
                osem.at[osel])
            return carry

        lax.fori_loop(0, _F, body, 0)
        owait(_F - 2, 0)
        owait(_F - 1, 1)

    return k


def kernel(x, cand, embedding):
    table = _retile(embedding.T)
    out3 = _build_gather()(x.T, cand.T, table)
    return out3.transpose(2, 0, 1)


# speedup vs baseline: 1.0275x; 1.0275x over previous
import functools

import jax
import jax.numpy as jnp
from jax import lax
from jax.experimental import pallas as pl
from jax.experimental.pallas import tpu as pltpu
from jax.experimental.pallas import tpu_sc as plsc

_B = 4096
_F = 26
_D = 16
_V = 2_600_000
_LANES = 16


_T_BLK = 131072
_SB = _T_BLK // 16
_TSH = _T_BLK.bit_length() - 1
_SSH = _SB.bit_length() - 1


def _transpose_body(et_ref, out_ref):
    et = et_ref[...]
    lo = jax.lax.bitcast_convert_type(
        et[:8].astype(jnp.bfloat16), jnp.uint16).astype(jnp.uint32)
    hi = jax.lax.bitcast_convert_type(
        et[8:].astype(jnp.bfloat16), jnp.uint16).astype(jnp.uint32)
    p = (lo | (hi << 16)).astype(jnp.int32)
    u = jnp.concatenate(
        [p[:, s * _SB:(s + 1) * _SB] for s in range(16)], axis=0)
    out_ref[...] = u.T


def _retile(et):
    grid = (_V + _T_BLK - 1) // _T_BLK
    return pl.pallas_call(
        _transpose_body,
        grid=(grid,),
        in_specs=[pl.BlockSpec((_D, _T_BLK), lambda i: (0, i))],
        out_specs=pl.BlockSpec((_SB, 128), lambda i: (i, 0)),
        out_shape=jax.ShapeDtypeStruct((grid * _SB, 128), jnp.int32),
    )(et)


def _build_gather():
    mesh = plsc.VectorSubcoreMesh(core_axis_name="c", subcore_axis_name="s")

    @functools.partial(
        pl.kernel,
        mesh=mesh,
        out_type=jax.ShapeDtypeStruct((_F, _D, _B), jnp.float32),
        compiler_params=pltpu.CompilerParams(needs_layout_passes=False),
        scratch_types=[
            pltpu.VMEM((_F, 128), jnp.int32),
            pltpu.VMEM((_F, 128), jnp.int32),
            pltpu.VMEM((6, 128), jnp.int32),
            pltpu.VMEM((6, 128, 128), jnp.int32),
            pltpu.VMEM((2, _D, 128), jnp.float32),
            pltpu.SemaphoreType.DMA((6,)),
            pltpu.SemaphoreType.DMA((2,)),
        ],
    )
    def k(xt_hbm, ct_hbm, table_hbm, out_hbm,
          xv, cv, bidx_v, blocks_v, outt_v, gsem, osem):
        wid = lax.axis_index("s") * 2 + lax.axis_index("c")
        b0 = wid * 128

        pltpu.sync_copy(xt_hbm.at[:, pl.ds(b0, 128)], xv)
        pltpu.sync_copy(ct_hbm.at[:, pl.ds(b0, 128)], cv)

        def compute_bidx(f, sel):
            def bb(j, carry):
                st = pl.multiple_of(j * _LANES, _LANES)
                xx = xv[f, pl.ds(st, _LANES)]
                bidx_v[sel, pl.ds(st, _LANES)] = (
                    ((xx >> _TSH) << _SSH) | (xx & (_SB - 1)))
                return carry
            lax.fori_loop(0, 128 // _LANES, bb, 0)

        def fire(f, sel):
            compute_bidx(f, sel)
            pltpu.async_copy(
                table_hbm.at[bidx_v.at[sel]], blocks_v.at[sel], gsem.at[sel])

        def gwait(sel):
            pltpu.make_async_copy(
                table_hbm.at[bidx_v.at[sel]], blocks_v.at[sel], gsem.at[sel]
            ).wait()

        def owait(f, sel):
            pltpu.make_async_copy(
                outt_v.at[sel],
                out_hbm.at[f, :, pl.ds(b0, 128)],
                osem.at[sel],
            ).wait()

        for p in range(5):
            fire(p, p)
        lanes = lax.iota(jnp.int32, _LANES)

        def body(f, carry):
            sel = lax.rem(f, 6)
            osel = lax.rem(f, 2)

            @pl.when(f < _F - 5)
            def _():
                fire(f + 5, lax.rem(f + 5, 6))

            gwait(sel)

            @pl.when(f >= 2)
            def _():
                owait(f - 2, osel)

            def kb(kk, carry2):
                st = pl.multiple_of(kk * _LANES, _LANES)
                x16 = xv[f, pl.ds(st, _LANES)]
                c16 = cv[f, pl.ds(st, _LANES)]
                off16 = ((x16 >> _SSH) & 15) << 3
                row16 = lanes + st
                sel16 = jnp.full((_LANES,), sel, jnp.int32)
                for dd in range(8):
                    v32 = plsc.load_gather(
                        blocks_v, [sel16, row16, off16 + dd])
                    flo = plsc.bitcast(v32 << 16, jnp.float32)
                    fhi = plsc.bitcast(v32 & jnp.int32(-65536), jnp.float32)
                    outt_v[osel, dd, pl.ds(st, _LANES)] = jnp.where(
                        c16 >= dd, flo, 0.0)
                    outt_v[osel, dd + 8, pl.ds(st, _LANES)] = jnp.where(
                        c16 >= dd + 8, fhi, 0.0)
                return carry2

            lax.fori_loop(0, 128 // _LANES, kb, 0)

            pltpu.async_copy(
                outt_v.at[osel], out_hbm.at[f, :, pl.ds(b0, 128)],
                osem.at[osel])
            return carry

        lax.fori_loop(0, _F, body, 0)
        owait(_F - 2, 0)
        owait(_F - 1, 1)

    return k


def kernel(x, cand, embedding):
    table = _retile(embedding.T)
    out3 = _build_gather()(x.T, cand.T, table)
    return out3.transpose(2, 0, 1)
